# Initial kernel scaffold; baseline (speedup 1.0000x reference)
import jax
import jax.numpy as jnp
from jax.experimental import pallas as pl


def kernel(anchors, cls_scores, box_deltas):
    raise NotImplementedError("write your pallas kernel here")


# speedup vs baseline: 2.0093x; 2.0093x over previous
import functools

import jax
import jax.numpy as jnp
from jax.experimental import pallas as pl

NUM_CLASSES = 80
IMAGE_HW = (512.0, 512.0)
IOU_TH = 0.5
PRE_K = 1000
POST_K = 100
SCORE_TH = 0.05
MIN_H, MIN_W = 2.0, 2.0
P = 1024


def _decode_mask_kernel(a_ref, s_ref, d_ref, masked_ref, cls_ref, box_ref):
    a = a_ref[0]
    ax1, ay1, ax2, ay2 = a[0], a[1], a[2], a[3]
    aw = ax2 - ax1
    ah = ay2 - ay1
    axc = ax1 + 0.5 * aw
    ayc = ay1 + 0.5 * ah
    d = d_ref[0]
    dx, dy = d[0], d[1]
    dw = jnp.clip(d[2], -4.0, 4.0)
    dh = jnp.clip(d[3], -4.0, 4.0)
    cx = axc + dx * aw
    cy = ayc + dy * ah
    w = aw * jnp.exp(dw)
    h = ah * jnp.exp(dh)
    Hh, Ww = IMAGE_HW
    x1 = jnp.clip(cx - 0.5 * w, 0.0, Ww)
    y1 = jnp.clip(cy - 0.5 * h, 0.0, Hh)
    x2 = jnp.clip(cx + 0.5 * w, 0.0, Ww)
    y2 = jnp.clip(cy + 0.5 * h, 0.0, Hh)
    box_ref[0] = jnp.stack([x1, y1, x2, y2], axis=0)

    scores = s_ref[0]
    mx = jnp.max(scores, axis=1)
    idx = jax.lax.broadcasted_iota(jnp.int32, scores.shape, 1)
    cand = jnp.where(scores == mx[:, None], idx, NUM_CLASSES)
    cls = jnp.min(cand, axis=1)
    cls_ref[0, 0] = cls

    valid = (mx > SCORE_TH) & ((y2 - y1) > MIN_H) & ((x2 - x1) > MIN_W)
    masked_ref[0, 0] = jnp.where(valid, mx, -1e9)


def _nms_kernel(s_ref, b_ref, out_ref):
    s = s_ref[0, 0]
    b = b_ref[0]
    x1, y1, x2, y2 = b[0], b[1], b[2], b[3]
    area = jnp.maximum(x2 - x1, 0.0) * jnp.maximum(y2 - y1, 0.0)
    ii = jax.lax.broadcasted_iota(jnp.int32, (P,), 0)

    def body(i, keep):
        m = (ii == i).astype(jnp.float32)
        x1i = jnp.sum(x1 * m)
        y1i = jnp.sum(y1 * m)
        x2i = jnp.sum(x2 * m)
        y2i = jnp.sum(y2 * m)
        si = jnp.sum(s * m)
        ki = jnp.max(keep * m)
        ai = jnp.maximum(x2i - x1i, 0.0) * jnp.maximum(y2i - y1i, 0.0)
        iw = jnp.maximum(jnp.minimum(x2, x2i) - jnp.maximum(x1, x1i), 0.0)
        ih = jnp.maximum(jnp.minimum(y2, y2i) - jnp.maximum(y1, y1i), 0.0)
        inter = iw * ih
        union = area + ai - inter
        iou = inter / jnp.maximum(union, 1e-6)
        sup = (iou > IOU_TH) & ((si - s) > 0.0) & (ii > i) & (ki > 0.0)
        return jnp.where(sup, 0.0, keep)

    keep = jax.lax.fori_loop(0, PRE_K, body, jnp.ones((P,), jnp.float32))
    out_ref[0, 0] = jnp.where((keep > 0.0) & (s > -1e8), s, -jnp.inf)


def kernel(anchors, cls_scores, box_deltas):
    B, N, C = cls_scores.shape
    BN = 2048
    NBLK = -(-N // BN)
    Np = NBLK * BN
    pad = Np - N
    anchors_p = jnp.pad(anchors, ((0, pad), (0, 0)))
    scores_p = jnp.pad(cls_scores, ((0, 0), (0, pad), (0, 0)),
                       constant_values=-1e9)
    deltas_p = jnp.pad(box_deltas, ((0, 0), (0, pad), (0, 0)))
    anchors_t = anchors_p.T[None]
    deltas_t = jnp.transpose(deltas_p, (0, 2, 1))

    masked, cls, boxes_t = pl.pallas_call(
        _decode_mask_kernel,
        grid=(B, NBLK),
        in_specs=[
            pl.BlockSpec((1, 4, BN), lambda b, j: (0, 0, j)),
            pl.BlockSpec((1, BN, C), lambda b, j: (b, j, 0)),
            pl.BlockSpec((1, 4, BN), lambda b, j: (b, 0, j)),
        ],
        out_specs=[
            pl.BlockSpec((1, 1, BN), lambda b, j: (b, 0, j)),
            pl.BlockSpec((1, 1, BN), lambda b, j: (b, 0, j)),
            pl.BlockSpec((1, 4, BN), lambda b, j: (b, 0, j)),
        ],
        out_shape=[
            jax.ShapeDtypeStruct((B, 1, Np), jnp.float32),
            jax.ShapeDtypeStruct((B, 1, Np), jnp.int32),
            jax.ShapeDtypeStruct((B, 4, Np), jnp.float32),
        ],
    )(anchors_t, scores_p, deltas_t)
    masked = masked[:, 0]
    cls = cls[:, 0]

    top_s, top_i = jax.lax.top_k(masked, PRE_K)
    b_sel = jnp.take_along_axis(boxes_t, top_i[:, None, :], axis=2)
    c_sel = jnp.take_along_axis(cls, top_i, axis=1)

    pad_n = P - PRE_K
    top_s_p = jnp.concatenate(
        [top_s, jnp.full((B, pad_n), -1e9, jnp.float32)], axis=1)
    b_p = jnp.concatenate(
        [b_sel, jnp.zeros((B, 4, pad_n), jnp.float32)], axis=2)
    c_p = jnp.concatenate(
        [c_sel, jnp.zeros((B, pad_n), jnp.int32)], axis=1)

    final = pl.pallas_call(
        _nms_kernel,
        grid=(B,),
        in_specs=[
            pl.BlockSpec((1, 1, P), lambda b: (b, 0, 0)),
            pl.BlockSpec((1, 4, P), lambda b: (b, 0, 0)),
        ],
        out_specs=pl.BlockSpec((1, 1, P), lambda b: (b, 0, 0)),
        out_shape=jax.ShapeDtypeStruct((B, 1, P), jnp.float32),
    )(top_s_p[:, None], b_p)
    final = final[:, 0]

    out_s, sel = jax.lax.top_k(final, POST_K)
    out_b = jnp.take_along_axis(b_p, sel[:, None, :], axis=2)
    out_b = jnp.transpose(out_b, (0, 2, 1))
    out_c = jnp.take_along_axis(c_p, sel, axis=1)

    pad = jnp.isfinite(out_s)
    out_b = jnp.where(pad[:, :, None], out_b, 0.0)
    out_s = jnp.where(pad, out_s, 0.0)
    out_c = jnp.where(pad, out_c, 0)
    return out_b, out_s, out_c


# speedup vs baseline: 3.6730x; 1.8281x over previous
import functools

import jax
import jax.numpy as jnp
from jax.experimental import pallas as pl

NUM_CLASSES = 80
IMAGE_HW = (512.0, 512.0)
IOU_TH = 0.5
PRE_K = 1000
POST_K = 100
SCORE_TH = 0.05
MIN_H, MIN_W = 2.0, 2.0
P = 1024


def _decode_mask_kernel(a_ref, s_ref, d_ref, masked_ref, cls_ref, box_ref):
    a = a_ref[0]
    ax1, ay1, ax2, ay2 = a[0], a[1], a[2], a[3]
    aw = ax2 - ax1
    ah = ay2 - ay1
    axc = ax1 + 0.5 * aw
    ayc = ay1 + 0.5 * ah
    d = d_ref[0]
    dx, dy = d[0], d[1]
    dw = jnp.clip(d[2], -4.0, 4.0)
    dh = jnp.clip(d[3], -4.0, 4.0)
    cx = axc + dx * aw
    cy = ayc + dy * ah
    w = aw * jnp.exp(dw)
    h = ah * jnp.exp(dh)
    Hh, Ww = IMAGE_HW
    x1 = jnp.clip(cx - 0.5 * w, 0.0, Ww)
    y1 = jnp.clip(cy - 0.5 * h, 0.0, Hh)
    x2 = jnp.clip(cx + 0.5 * w, 0.0, Ww)
    y2 = jnp.clip(cy + 0.5 * h, 0.0, Hh)
    box_ref[0] = jnp.stack([x1, y1, x2, y2], axis=0)

    scores = s_ref[0]
    mx = jnp.max(scores, axis=1)
    idx = jax.lax.broadcasted_iota(jnp.int32, scores.shape, 1)
    cand = jnp.where(scores == mx[:, None], idx, NUM_CLASSES)
    cls = jnp.min(cand, axis=1)
    cls_ref[0, 0] = cls

    valid = (mx > SCORE_TH) & ((y2 - y1) > MIN_H) & ((x2 - x1) > MIN_W)
    masked_ref[0, 0] = jnp.where(valid, mx, -1e9)


def _nms_kernel(s_ref, b_ref, out_ref):
    s = s_ref[...]
    x1 = b_ref[:, 0]
    y1 = b_ref[:, 1]
    x2 = b_ref[:, 2]
    y2 = b_ref[:, 3]
    area = jnp.maximum(x2 - x1, 0.0) * jnp.maximum(y2 - y1, 0.0)
    ii = jax.lax.broadcasted_iota(jnp.int32, (1, P), 1)

    def body(i, keep):
        m = (ii == i).astype(jnp.float32)
        x1i = jnp.sum(x1 * m, axis=1, keepdims=True)
        y1i = jnp.sum(y1 * m, axis=1, keepdims=True)
        x2i = jnp.sum(x2 * m, axis=1, keepdims=True)
        y2i = jnp.sum(y2 * m, axis=1, keepdims=True)
        si = jnp.sum(s * m, axis=1, keepdims=True)
        ki = jnp.max(keep * m, axis=1, keepdims=True)
        ai = jnp.maximum(x2i - x1i, 0.0) * jnp.maximum(y2i - y1i, 0.0)
        iw = jnp.maximum(jnp.minimum(x2, x2i) - jnp.maximum(x1, x1i), 0.0)
        ih = jnp.maximum(jnp.minimum(y2, y2i) - jnp.maximum(y1, y1i), 0.0)
        inter = iw * ih
        union = area + ai - inter
        iou = inter / jnp.maximum(union, 1e-6)
        sup = (iou > IOU_TH) & ((si - s) > 0.0) & (ii > i) & (ki > 0.0)
        return jnp.where(sup, 0.0, keep)

    keep = jax.lax.fori_loop(0, PRE_K, body, jnp.ones(s.shape, jnp.float32))
    out_ref[...] = jnp.where((keep > 0.0) & (s > -1e8), s, -jnp.inf)


def kernel(anchors, cls_scores, box_deltas):
    B, N, C = cls_scores.shape
    BN = 2048
    NBLK = -(-N // BN)
    Np = NBLK * BN
    pad = Np - N
    anchors_p = jnp.pad(anchors, ((0, pad), (0, 0)))
    scores_p = jnp.pad(cls_scores, ((0, 0), (0, pad), (0, 0)),
                       constant_values=-1e9)
    deltas_p = jnp.pad(box_deltas, ((0, 0), (0, pad), (0, 0)))
    anchors_t = anchors_p.T[None]
    deltas_t = jnp.transpose(deltas_p, (0, 2, 1))

    masked, cls, boxes_t = pl.pallas_call(
        _decode_mask_kernel,
        grid=(B, NBLK),
        in_specs=[
            pl.BlockSpec((1, 4, BN), lambda b, j: (0, 0, j)),
            pl.BlockSpec((1, BN, C), lambda b, j: (b, j, 0)),
            pl.BlockSpec((1, 4, BN), lambda b, j: (b, 0, j)),
        ],
        out_specs=[
            pl.BlockSpec((1, 1, BN), lambda b, j: (b, 0, j)),
            pl.BlockSpec((1, 1, BN), lambda b, j: (b, 0, j)),
            pl.BlockSpec((1, 4, BN), lambda b, j: (b, 0, j)),
        ],
        out_shape=[
            jax.ShapeDtypeStruct((B, 1, Np), jnp.float32),
            jax.ShapeDtypeStruct((B, 1, Np), jnp.int32),
            jax.ShapeDtypeStruct((B, 4, Np), jnp.float32),
        ],
    )(anchors_t, scores_p, deltas_t)
    masked = masked[:, 0]
    cls = cls[:, 0]

    top_s, top_i = jax.lax.top_k(masked, PRE_K)
    b_sel = jnp.take_along_axis(boxes_t, top_i[:, None, :], axis=2)
    c_sel = jnp.take_along_axis(cls, top_i, axis=1)

    pad_n = P - PRE_K
    top_s_p = jnp.concatenate(
        [top_s, jnp.full((B, pad_n), -1e9, jnp.float32)], axis=1)
    b_p = jnp.concatenate(
        [b_sel, jnp.zeros((B, 4, pad_n), jnp.float32)], axis=2)
    c_p = jnp.concatenate(
        [c_sel, jnp.zeros((B, pad_n), jnp.int32)], axis=1)

    final = pl.pallas_call(
        _nms_kernel,
        out_shape=jax.ShapeDtypeStruct((B, P), jnp.float32),
    )(top_s_p, b_p)

    out_s, sel = jax.lax.top_k(final, POST_K)
    out_b = jnp.take_along_axis(b_p, sel[:, None, :], axis=2)
    out_b = jnp.transpose(out_b, (0, 2, 1))
    out_c = jnp.take_along_axis(c_p, sel, axis=1)

    pad = jnp.isfinite(out_s)
    out_b = jnp.where(pad[:, :, None], out_b, 0.0)
    out_s = jnp.where(pad, out_s, 0.0)
    out_c = jnp.where(pad, out_c, 0)
    return out_b, out_s, out_c


# speedup vs baseline: 3.6957x; 1.0062x over previous
import functools

import jax
import jax.numpy as jnp
from jax.experimental import pallas as pl

NUM_CLASSES = 80
IMAGE_HW = (512.0, 512.0)
IOU_TH = 0.5
PRE_K = 1000
POST_K = 100
SCORE_TH = 0.05
MIN_H, MIN_W = 2.0, 2.0
P = 1024


def _decode_mask_kernel(a_ref, s_ref, d_ref, masked_ref, cls_ref, box_ref):
    a = a_ref[0]
    ax1, ay1, ax2, ay2 = a[0], a[1], a[2], a[3]
    aw = ax2 - ax1
    ah = ay2 - ay1
    axc = ax1 + 0.5 * aw
    ayc = ay1 + 0.5 * ah
    d = d_ref[0]
    dx, dy = d[0], d[1]
    dw = jnp.clip(d[2], -4.0, 4.0)
    dh = jnp.clip(d[3], -4.0, 4.0)
    cx = axc + dx * aw
    cy = ayc + dy * ah
    w = aw * jnp.exp(dw)
    h = ah * jnp.exp(dh)
    Hh, Ww = IMAGE_HW
    x1 = jnp.clip(cx - 0.5 * w, 0.0, Ww)
    y1 = jnp.clip(cy - 0.5 * h, 0.0, Hh)
    x2 = jnp.clip(cx + 0.5 * w, 0.0, Ww)
    y2 = jnp.clip(cy + 0.5 * h, 0.0, Hh)
    box_ref[0] = jnp.stack([x1, y1, x2, y2], axis=0)

    scores = s_ref[0]
    mx = jnp.max(scores, axis=1)
    idx = jax.lax.broadcasted_iota(jnp.int32, scores.shape, 1)
    cand = jnp.where(scores == mx[:, None], idx, NUM_CLASSES)
    cls = jnp.min(cand, axis=1)
    cls_ref[0, 0] = cls

    valid = (mx > SCORE_TH) & ((y2 - y1) > MIN_H) & ((x2 - x1) > MIN_W)
    masked_ref[0, 0] = jnp.where(valid, mx, -1e9)


def _nms_kernel(s_ref, b_ref, out_ref):
    s = s_ref[...]
    x1 = b_ref[:, 0]
    y1 = b_ref[:, 1]
    x2 = b_ref[:, 2]
    y2 = b_ref[:, 3]
    area = jnp.maximum(x2 - x1, 0.0) * jnp.maximum(y2 - y1, 0.0)
    ii = jax.lax.broadcasted_iota(jnp.int32, (1, P), 1)
    vals = jnp.stack([x1, y1, x2, y2, s], axis=0)

    def body(i, keep):
        m = (ii == i).astype(jnp.float32)
        ext = jnp.sum(vals * m[None], axis=2, keepdims=True)
        x1i = ext[0]
        y1i = ext[1]
        x2i = ext[2]
        y2i = ext[3]
        si = ext[4]
        ki = jnp.max(keep * m, axis=1, keepdims=True)
        ai = jnp.maximum(x2i - x1i, 0.0) * jnp.maximum(y2i - y1i, 0.0)
        iw = jnp.maximum(jnp.minimum(x2, x2i) - jnp.maximum(x1, x1i), 0.0)
        ih = jnp.maximum(jnp.minimum(y2, y2i) - jnp.maximum(y1, y1i), 0.0)
        inter = iw * ih
        union = area + ai - inter
        iou = inter / jnp.maximum(union, 1e-6)
        sup = (iou > IOU_TH) & ((si - s) > 0.0) & (ii > i) & (ki > 0.0)
        return jnp.where(sup, 0.0, keep)

    keep = jax.lax.fori_loop(0, PRE_K, body, jnp.ones(s.shape, jnp.float32))
    out_ref[...] = jnp.where((keep > 0.0) & (s > -1e8), s, -jnp.inf)


def kernel(anchors, cls_scores, box_deltas):
    B, N, C = cls_scores.shape
    BN = 2048
    NBLK = -(-N // BN)
    Np = NBLK * BN
    pad = Np - N
    anchors_p = jnp.pad(anchors, ((0, pad), (0, 0)))
    scores_p = jnp.pad(cls_scores, ((0, 0), (0, pad), (0, 0)),
                       constant_values=-1e9)
    deltas_p = jnp.pad(box_deltas, ((0, 0), (0, pad), (0, 0)))
    anchors_t = anchors_p.T[None]
    deltas_t = jnp.transpose(deltas_p, (0, 2, 1))

    masked, cls, boxes_t = pl.pallas_call(
        _decode_mask_kernel,
        grid=(B, NBLK),
        in_specs=[
            pl.BlockSpec((1, 4, BN), lambda b, j: (0, 0, j)),
            pl.BlockSpec((1, BN, C), lambda b, j: (b, j, 0)),
            pl.BlockSpec((1, 4, BN), lambda b, j: (b, 0, j)),
        ],
        out_specs=[
            pl.BlockSpec((1, 1, BN), lambda b, j: (b, 0, j)),
            pl.BlockSpec((1, 1, BN), lambda b, j: (b, 0, j)),
            pl.BlockSpec((1, 4, BN), lambda b, j: (b, 0, j)),
        ],
        out_shape=[
            jax.ShapeDtypeStruct((B, 1, Np), jnp.float32),
            jax.ShapeDtypeStruct((B, 1, Np), jnp.int32),
            jax.ShapeDtypeStruct((B, 4, Np), jnp.float32),
        ],
    )(anchors_t, scores_p, deltas_t)
    masked = masked[:, 0]
    cls = cls[:, 0]

    top_s, top_i = jax.lax.top_k(masked, PRE_K)
    b_sel = jnp.take_along_axis(boxes_t, top_i[:, None, :], axis=2)
    c_sel = jnp.take_along_axis(cls, top_i, axis=1)

    pad_n = P - PRE_K
    top_s_p = jnp.concatenate(
        [top_s, jnp.full((B, pad_n), -1e9, jnp.float32)], axis=1)
    b_p = jnp.concatenate(
        [b_sel, jnp.zeros((B, 4, pad_n), jnp.float32)], axis=2)
    c_p = jnp.concatenate(
        [c_sel, jnp.zeros((B, pad_n), jnp.int32)], axis=1)

    final = pl.pallas_call(
        _nms_kernel,
        out_shape=jax.ShapeDtypeStruct((B, P), jnp.float32),
    )(top_s_p, b_p)

    out_s, sel = jax.lax.top_k(final, POST_K)
    out_b = jnp.take_along_axis(b_p, sel[:, None, :], axis=2)
    out_b = jnp.transpose(out_b, (0, 2, 1))
    out_c = jnp.take_along_axis(c_p, sel, axis=1)

    pad = jnp.isfinite(out_s)
    out_b = jnp.where(pad[:, :, None], out_b, 0.0)
    out_s = jnp.where(pad, out_s, 0.0)
    out_c = jnp.where(pad, out_c, 0)
    return out_b, out_s, out_c


# speedup vs baseline: 4.2146x; 1.1404x over previous
import functools

import jax
import jax.numpy as jnp
from jax.experimental import pallas as pl

NUM_CLASSES = 80
IMAGE_HW = (512.0, 512.0)
IOU_TH = 0.5
PRE_K = 1000
POST_K = 100
SCORE_TH = 0.05
MIN_H, MIN_W = 2.0, 2.0
P = 1024


def _decode_mask_kernel(a_ref, s_ref, d_ref, masked_ref, cls_ref, box_ref):
    a = a_ref[0]
    ax1, ay1, ax2, ay2 = a[0], a[1], a[2], a[3]
    aw = ax2 - ax1
    ah = ay2 - ay1
    axc = ax1 + 0.5 * aw
    ayc = ay1 + 0.5 * ah
    d = d_ref[0]
    dx, dy = d[0], d[1]
    dw = jnp.clip(d[2], -4.0, 4.0)
    dh = jnp.clip(d[3], -4.0, 4.0)
    cx = axc + dx * aw
    cy = ayc + dy * ah
    w = aw * jnp.exp(dw)
    h = ah * jnp.exp(dh)
    Hh, Ww = IMAGE_HW
    x1 = jnp.clip(cx - 0.5 * w, 0.0, Ww)
    y1 = jnp.clip(cy - 0.5 * h, 0.0, Hh)
    x2 = jnp.clip(cx + 0.5 * w, 0.0, Ww)
    y2 = jnp.clip(cy + 0.5 * h, 0.0, Hh)
    box_ref[0] = jnp.stack([x1, y1, x2, y2], axis=0)

    scores = s_ref[0]
    mx = jnp.max(scores, axis=1)
    idx = jax.lax.broadcasted_iota(jnp.int32, scores.shape, 1)
    cand = jnp.where(scores == mx[:, None], idx, NUM_CLASSES)
    cls = jnp.min(cand, axis=1)
    cls_ref[0, 0] = cls

    valid = (mx > SCORE_TH) & ((y2 - y1) > MIN_H) & ((x2 - x1) > MIN_W)
    masked_ref[0, 0] = jnp.where(valid, mx, -1e9)


def _nms_kernel(s_ref, b_ref, out_ref):
    s = s_ref[...]
    x1 = b_ref[:, 0]
    y1 = b_ref[:, 1]
    x2 = b_ref[:, 2]
    y2 = b_ref[:, 3]
    area = jnp.maximum(x2 - x1, 0.0) * jnp.maximum(y2 - y1, 0.0)
    ii = jax.lax.broadcasted_iota(jnp.int32, (1, P), 1)
    vals = jnp.stack([x1, y1, x2, y2, s], axis=0)

    def body(i, keep):
        m = (ii == i).astype(jnp.float32)
        ext = jnp.sum(vals * m[None], axis=2, keepdims=True)
        x1i = ext[0]
        y1i = ext[1]
        x2i = ext[2]
        y2i = ext[3]
        si = ext[4]
        ki = jnp.max(keep * m, axis=1, keepdims=True)
        ai = jnp.maximum(x2i - x1i, 0.0) * jnp.maximum(y2i - y1i, 0.0)
        iw = jnp.maximum(jnp.minimum(x2, x2i) - jnp.maximum(x1, x1i), 0.0)
        ih = jnp.maximum(jnp.minimum(y2, y2i) - jnp.maximum(y1, y1i), 0.0)
        inter = iw * ih
        union = area + ai - inter
        iou = inter / jnp.maximum(union, 1e-6)
        sup = (iou > IOU_TH) & ((si - s) > 0.0) & (ii > i) & (ki > 0.0)
        return jnp.where(sup, 0.0, keep)

    keep = jax.lax.fori_loop(0, PRE_K, body, jnp.ones(s.shape, jnp.float32))
    out_ref[...] = jnp.where((keep > 0.0) & (s > -1e8), s, -jnp.inf)


def kernel(anchors, cls_scores, box_deltas):
    B, N, C = cls_scores.shape
    BN = 2048
    NBLK = -(-N // BN)
    anchors_t = anchors.T[None]
    deltas_t = jnp.transpose(box_deltas, (0, 2, 1))

    masked, cls, boxes_t = pl.pallas_call(
        _decode_mask_kernel,
        grid=(B, NBLK),
        in_specs=[
            pl.BlockSpec((1, 4, BN), lambda b, j: (0, 0, j)),
            pl.BlockSpec((1, BN, C), lambda b, j: (b, j, 0)),
            pl.BlockSpec((1, 4, BN), lambda b, j: (b, 0, j)),
        ],
        out_specs=[
            pl.BlockSpec((1, 1, BN), lambda b, j: (b, 0, j)),
            pl.BlockSpec((1, 1, BN), lambda b, j: (b, 0, j)),
            pl.BlockSpec((1, 4, BN), lambda b, j: (b, 0, j)),
        ],
        out_shape=[
            jax.ShapeDtypeStruct((B, 1, N), jnp.float32),
            jax.ShapeDtypeStruct((B, 1, N), jnp.int32),
            jax.ShapeDtypeStruct((B, 4, N), jnp.float32),
        ],
    )(anchors_t, cls_scores, deltas_t)
    masked = masked[:, 0]
    cls = cls[:, 0]

    top_s, top_i = jax.lax.top_k(masked, PRE_K)
    b_sel = jnp.take_along_axis(boxes_t, top_i[:, None, :], axis=2)
    c_sel = jnp.take_along_axis(cls, top_i, axis=1)

    pad_n = P - PRE_K
    top_s_p = jnp.concatenate(
        [top_s, jnp.full((B, pad_n), -1e9, jnp.float32)], axis=1)
    b_p = jnp.concatenate(
        [b_sel, jnp.zeros((B, 4, pad_n), jnp.float32)], axis=2)
    c_p = jnp.concatenate(
        [c_sel, jnp.zeros((B, pad_n), jnp.int32)], axis=1)

    final = pl.pallas_call(
        _nms_kernel,
        out_shape=jax.ShapeDtypeStruct((B, P), jnp.float32),
    )(top_s_p, b_p)

    out_s, sel = jax.lax.top_k(final, POST_K)
    out_b = jnp.take_along_axis(b_p, sel[:, None, :], axis=2)
    out_b = jnp.transpose(out_b, (0, 2, 1))
    out_c = jnp.take_along_axis(c_p, sel, axis=1)

    pad = jnp.isfinite(out_s)
    out_b = jnp.where(pad[:, :, None], out_b, 0.0)
    out_s = jnp.where(pad, out_s, 0.0)
    out_c = jnp.where(pad, out_c, 0)
    return out_b, out_s, out_c
